# baseline (speedup 1.0000x reference)
import functools

import jax
import jax.numpy as jnp
from jax import lax
from jax.experimental import pallas as pl
from jax.experimental.pallas import tpu as pltpu
from jax.experimental.pallas import tpu_sc as plsc

NJ = 50
NI = 16384
DIM = 64
NC, NS = 2, 16
NW = NC * NS
LANES = 128
NGROUPS = NJ * (NI // LANES)
GPW = NGROUPS // NW

_mesh = plsc.VectorSubcoreMesh(core_axis_name="c", subcore_axis_name="s")


@functools.partial(
    pl.kernel,
    out_type=jax.ShapeDtypeStruct((NJ, DIM, NI), jnp.float32),
    mesh=_mesh,
    scratch_types=[
        pltpu.VMEM((LANES,), jnp.int32),
        pltpu.VMEM((2, LANES, LANES), jnp.float32),
        pltpu.VMEM((2, LANES), jnp.int32),
        pltpu.VMEM((8, LANES), jnp.float32),
        pltpu.VMEM((2, LANES), jnp.int32),
        pltpu.SemaphoreType.DMA,
    ],
    compiler_params=pltpu.CompilerParams(
        use_tc_tiling_on_sc=True, needs_layout_passes=False),
)
def _emb_lookup(ids_hbm, table_hbm, out_hbm, idsv, gbuf, gidx, otile, cbase,
                gsem):
    wid = lax.axis_index("s") * NC + lax.axis_index("c")
    g0 = wid * GPW
    iota = lax.iota(jnp.int32, 16)

    def load_ids_and_fire(item, b):
        j = item // (NI // LANES)
        c = item % (NI // LANES)
        pltpu.sync_copy(ids_hbm.at[j, pl.ds(c * LANES, LANES)], idsv)
        for k in range(8):
            v = idsv[pl.ds(16 * k, 16)]
            gidx.at[b][pl.ds(16 * k, 16)] = lax.shift_right_logical(v, 1)
            cbase.at[b][pl.ds(16 * k, 16)] = lax.shift_left(
                lax.bitwise_and(v, 1), 6)
        pltpu.async_copy(table_hbm.at[gidx.at[b]], gbuf.at[b], gsem)

    def wait_gather(b):
        pltpu.make_async_copy(
            table_hbm.at[pl.ds(0, LANES)], gbuf.at[b], gsem).wait()

    def transpose_and_store(item, b):
        j = item // (NI // LANES)
        c = item % (NI // LANES)
        G = gbuf.at[b]
        cb = [cbase.at[b][pl.ds(16 * lb, 16)] for lb in range(8)]
        li = [iota + (16 * lb) for lb in range(8)]

        @pl.loop(0, 8)
        def _r(r):
            for s in range(8):
                d = r * 8 + s
                for lb in range(8):
                    val = plsc.load_gather(G, [li[lb], cb[lb] + d])
                    otile[s, pl.ds(16 * lb, 16)] = val
            pltpu.sync_copy(
                otile,
                out_hbm.at[j, pl.ds(r * 8, 8), pl.ds(c * LANES, LANES)])

    load_ids_and_fire(g0, 0)

    @pl.loop(0, GPW, step=2)
    def _g(g):
        item = g0 + g
        wait_gather(0)
        load_ids_and_fire(item + 1, 1)
        transpose_and_store(item, 0)
        wait_gather(1)

        @pl.when(g + 2 < GPW)
        def _():
            load_ids_and_fire(item + 2, 0)
        transpose_and_store(item + 1, 1)


def kernel(token_ids, W):
    ids_t = token_ids.T.astype(jnp.int32)
    table = W.reshape(500000, 128)
    out = _emb_lookup(ids_t, table)
    return out.transpose(2, 0, 1)


# speedup vs baseline: 1.1091x; 1.1091x over previous
import functools

import jax
import jax.numpy as jnp
from jax import lax
from jax.experimental import pallas as pl
from jax.experimental.pallas import tpu as pltpu
from jax.experimental.pallas import tpu_sc as plsc

NUM_TOKENS = 16384 * 50
DIM = 64
NJ, NI = 50, 16384
NC, NS = 2, 16
NW = NC * NS
IPG = 128
ROWS_PER_WORKER = NUM_TOKENS // (NW * IPG)
C_ROWS = 5
CHUNK_IDS = C_ROWS * IPG
NCHUNKS = ROWS_PER_WORKER // C_ROWS
NPAIR = NCHUNKS // 2

_mesh = plsc.VectorSubcoreMesh(core_axis_name="c", subcore_axis_name="s")


@functools.partial(
    pl.kernel,
    out_type=jax.ShapeDtypeStruct((NUM_TOKENS, DIM), jnp.float32),
    mesh=_mesh,
    scratch_types=[
        pltpu.VMEM((ROWS_PER_WORKER, IPG), jnp.int32),
        pltpu.VMEM((2, CHUNK_IDS, DIM), jnp.float32),
        pltpu.SemaphoreType.DMA,
        pltpu.SemaphoreType.DMA,
        pltpu.SemaphoreType.DMA,
        pltpu.SemaphoreType.DMA,
    ],
    compiler_params=pltpu.CompilerParams(use_tc_tiling_on_sc=False),
)
def _gather_sc(idx_hbm, table_hbm, out_hbm, idx_v, rows_v, gsem0, gsem1,
               ssem0, ssem1):
    wid = lax.axis_index("s") * NC + lax.axis_index("c")
    row0 = wid * ROWS_PER_WORKER
    out0 = row0 * IPG

    pltpu.sync_copy(idx_hbm.at[pl.ds(row0, ROWS_PER_WORKER)], idx_v)

    def fire_gathers(c, buf, sem):
        for j in range(C_ROWS):
            pltpu.async_copy(
                table_hbm.at[idx_v.at[c * C_ROWS + j]],
                rows_v.at[buf].at[pl.ds(j * IPG, IPG)],
                sem,
            )

    def wait_gathers(buf, sem):
        for j in range(C_ROWS):
            pltpu.make_async_copy(
                table_hbm.at[pl.ds(0, IPG)],
                rows_v.at[buf].at[pl.ds(j * IPG, IPG)],
                sem,
            ).wait()

    def fire_store(c, buf, sem):
        pltpu.async_copy(
            rows_v.at[buf], out_hbm.at[pl.ds(out0 + c * CHUNK_IDS, CHUNK_IDS)],
            sem,
        )

    def wait_store(buf, sem):
        pltpu.make_async_copy(
            rows_v.at[buf], out_hbm.at[pl.ds(0, CHUNK_IDS)], sem,
        ).wait()

    fire_gathers(0, 0, gsem0)

    @pl.loop(0, NPAIR)
    def _pair(k):
        i = 2 * k

        @pl.when(k > 0)
        def _():
            wait_store(1, ssem1)
        fire_gathers(i + 1, 1, gsem1)

        wait_gathers(0, gsem0)
        fire_store(i, 0, ssem0)

        @pl.when(k < NPAIR - 1)
        def _():
            wait_store(0, ssem0)
            fire_gathers(i + 2, 0, gsem0)

        wait_gathers(1, gsem1)
        fire_store(i + 1, 1, ssem1)

    wait_store(0, ssem0)
    wait_store(1, ssem1)


@functools.partial(
    pl.pallas_call,
    grid=(NJ * NI // 512,),
    in_specs=[pl.BlockSpec((256, 128), lambda b: (b, 0))],
    out_specs=pl.BlockSpec((1, DIM, 512), lambda b: (b // 32, 0, b % 32)),
    out_shape=jax.ShapeDtypeStruct((NJ, DIM, NI), jnp.float32),
)
def _retile_tc(y_ref, o_ref):
    x = y_ref[...]
    lo = x[:, :DIM].T
    hi = x[:, DIM:].T
    o_ref[...] = jnp.concatenate([lo, hi], axis=1)[None]


def kernel(token_ids, W):
    ids_flat = token_ids.T.reshape(-1).astype(jnp.int32)
    ids_perm = ids_flat.reshape(-1, 2, 256).transpose(0, 2, 1)
    rows = _gather_sc(ids_perm.reshape(NUM_TOKENS // IPG, IPG), W)
    out = _retile_tc(rows.reshape(NUM_TOKENS // 2, 2 * DIM))
    return out.transpose(2, 0, 1)


# speedup vs baseline: 1.3761x; 1.2407x over previous
import functools

import jax
import jax.numpy as jnp
from jax import lax
from jax.experimental import pallas as pl
from jax.experimental.pallas import tpu as pltpu
from jax.experimental.pallas import tpu_sc as plsc

NUM_TOKENS = 16384 * 50
DIM = 64
NJ, NI = 50, 16384
NC, NS = 2, 16
NW = NC * NS
IPG = 128
ROWS_PER_WORKER = NUM_TOKENS // (NW * IPG)
C_ROWS = 5
CHUNK_IDS = C_ROWS * IPG
NCHUNKS = ROWS_PER_WORKER // C_ROWS
NPAIR = NCHUNKS // 2

_mesh = plsc.VectorSubcoreMesh(core_axis_name="c", subcore_axis_name="s")


@functools.partial(
    pl.kernel,
    out_type=jax.ShapeDtypeStruct((NUM_TOKENS, DIM), jnp.float32),
    mesh=_mesh,
    scratch_types=[
        pltpu.VMEM((ROWS_PER_WORKER, IPG), jnp.int32),
        pltpu.VMEM((2, CHUNK_IDS, DIM), jnp.float32),
        pltpu.SemaphoreType.DMA,
        pltpu.SemaphoreType.DMA,
        pltpu.SemaphoreType.DMA,
        pltpu.SemaphoreType.DMA,
    ],
    compiler_params=pltpu.CompilerParams(use_tc_tiling_on_sc=False),
)
def _gather_sc(idx_hbm, table_hbm, out_hbm, idx_v, rows_v, gsem0, gsem1,
               ssem0, ssem1):
    wid = lax.axis_index("s") * NC + lax.axis_index("c")
    row0 = wid * ROWS_PER_WORKER
    out0 = row0 * IPG

    pltpu.sync_copy(idx_hbm.at[pl.ds(row0, ROWS_PER_WORKER)], idx_v)

    def fire_gathers(c, buf, sem):
        for j in range(C_ROWS):
            pltpu.async_copy(
                table_hbm.at[idx_v.at[c * C_ROWS + j]],
                rows_v.at[buf].at[pl.ds(j * IPG, IPG)],
                sem,
            )

    def wait_gathers(buf, sem):
        for j in range(C_ROWS):
            pltpu.make_async_copy(
                table_hbm.at[pl.ds(0, IPG)],
                rows_v.at[buf].at[pl.ds(j * IPG, IPG)],
                sem,
            ).wait()

    def fire_store(c, buf, sem):
        pltpu.async_copy(
            rows_v.at[buf], out_hbm.at[pl.ds(out0 + c * CHUNK_IDS, CHUNK_IDS)],
            sem,
        )

    def wait_store(buf, sem):
        pltpu.make_async_copy(
            rows_v.at[buf], out_hbm.at[pl.ds(0, CHUNK_IDS)], sem,
        ).wait()

    fire_gathers(0, 0, gsem0)

    @pl.loop(0, NPAIR)
    def _pair(k):
        i = 2 * k

        @pl.when(k > 0)
        def _():
            wait_store(1, ssem1)
        fire_gathers(i + 1, 1, gsem1)

        wait_gathers(0, gsem0)
        fire_store(i, 0, ssem0)

        @pl.when(k < NPAIR - 1)
        def _():
            wait_store(0, ssem0)
            fire_gathers(i + 2, 0, gsem0)

        wait_gathers(1, gsem1)
        fire_store(i + 1, 1, ssem1)

    wait_store(0, ssem0)
    wait_store(1, ssem1)


_TCB = 512


@functools.partial(
    pl.pallas_call,
    grid=(NJ * NI // (2 * _TCB),),
    in_specs=[pl.BlockSpec((_TCB, 128), lambda b: (b, 0))],
    out_specs=pl.BlockSpec(
        (1, DIM, 2 * _TCB), lambda b: (b // (NI // (2 * _TCB)), 0,
                                       b % (NI // (2 * _TCB)))),
    out_shape=jax.ShapeDtypeStruct((NJ, DIM, NI), jnp.float32),
)
def _retile_tc(y_ref, o_ref):
    x = y_ref[...]
    eye = jnp.eye(_TCB, dtype=jnp.float32)
    dims = (((0,), (0,)), ((), ()))
    lo = lax.dot_general(x[:, :DIM], eye, dims,
                         preferred_element_type=jnp.float32)
    hi = lax.dot_general(x[:, DIM:], eye, dims,
                         preferred_element_type=jnp.float32)
    o_ref[...] = jnp.concatenate([lo, hi], axis=1)[None]


def kernel(token_ids, W):
    ids_flat = token_ids.T.reshape(-1).astype(jnp.int32)
    ids_perm = ids_flat.reshape(-1, 2, _TCB).transpose(0, 2, 1)
    rows = _gather_sc(ids_perm.reshape(NUM_TOKENS // IPG, IPG), W)
    out = _retile_tc(rows.reshape(NUM_TOKENS // 2, 2 * DIM))
    return out.transpose(2, 0, 1)
